# baseline (speedup 1.0000x reference)
import functools

import jax
import jax.numpy as jnp
from jax import lax
from jax.experimental import pallas as pl
from jax.experimental.pallas import tpu as pltpu
from jax.experimental.pallas import tpu_sc as plsc

N_NODES = 10000
IN_DIM = 128
HID = 64
HEADS = 8
NCLS = 40

N_PAD = 10240
NR = N_PAD // 128
C2 = 48
CHUNK = 128
NSC = 2
NTILES = 16
SLAB = N_PAD // NTILES
BLKSUB = 56
BLK = BLKSUB * CHUNK
MCH = 64

_f32 = jnp.float32
_i32 = jnp.int32

_SC_PARAMS = pltpu.CompilerParams(
    needs_layout_passes=False, use_tc_tiling_on_sc=False)


def _iota16():
    return lax.iota(_i32, 16)


def _leaky(e):
    return jnp.maximum(e, 0.2 * e)


_BN1 = 1024


def _tc1_body(x_ref, w_ref, aw_ref, h_ref, asad_ref):
    h = jnp.dot(x_ref[...], w_ref[...], preferred_element_type=_f32)
    h_ref[...] = h
    asad_ref[...] = lax.dot_general(
        aw_ref[...], h, (((0,), (1,)), ((), ())), preferred_element_type=_f32)


def _tc1(x_p, w1, aw1):
    return pl.pallas_call(
        _tc1_body,
        grid=(N_PAD // _BN1,),
        in_specs=[
            pl.BlockSpec((_BN1, IN_DIM), lambda i: (i, 0)),
            pl.BlockSpec((IN_DIM, HEADS * HID), lambda i: (0, 0)),
            pl.BlockSpec((HEADS * HID, 2 * HEADS), lambda i: (0, 0)),
        ],
        out_specs=[
            pl.BlockSpec((_BN1, HEADS * HID), lambda i: (i, 0)),
            pl.BlockSpec((2 * HEADS, _BN1), lambda i: (0, i)),
        ],
        out_shape=[
            jax.ShapeDtypeStruct((N_PAD, HEADS * HID), _f32),
            jax.ShapeDtypeStruct((2 * HEADS, N_PAD), _f32),
        ],
    )(x_p, w1, aw1)


_BN2 = 1024


def _tc2_body(o1_ref, b1_ref, w2_ref, a2_ref, h2_ref, asad2_ref):
    acc = jnp.zeros((_BN2, C2), _f32)
    for h in range(HEADS):
        v = o1_ref[h] + b1_ref[h][None, :]
        emb_h = jnp.where(v > 0, v, jnp.exp(jnp.minimum(v, 0.0)) - 1.0)
        acc = acc + jnp.dot(emb_h, w2_ref[h], preferred_element_type=_f32)
    h2_ref[...] = acc
    a2 = a2_ref[...]
    s = jnp.sum(acc * a2[0][None, :], axis=1)
    d = jnp.sum(acc * a2[1][None, :], axis=1)
    asad2_ref[...] = jnp.concatenate([s[None, :], d[None, :]], axis=0)


def _tc2(o1, b1r, w2p, a2):
    return pl.pallas_call(
        _tc2_body,
        grid=(N_PAD // _BN2,),
        in_specs=[
            pl.BlockSpec((HEADS, _BN2, HID), lambda i: (0, i, 0)),
            pl.BlockSpec((HEADS, HID), lambda i: (0, 0)),
            pl.BlockSpec((HEADS, HID, C2), lambda i: (0, 0, 0)),
            pl.BlockSpec((2, C2), lambda i: (0, 0)),
        ],
        out_specs=[
            pl.BlockSpec((_BN2, C2), lambda i: (i, 0)),
            pl.BlockSpec((2, _BN2), lambda i: (0, i)),
        ],
        out_shape=[
            jax.ShapeDtypeStruct((N_PAD, C2), _f32),
            jax.ShapeDtypeStruct((2, N_PAD), _f32),
        ],
    )(o1, b1r, w2p, a2)


_BN3 = 2000


def _tc3_body(p_ref, b2_ref, out_ref):
    s = p_ref[0] + p_ref[1] + b2_ref[...]
    out_ref[...] = s[:, :NCLS]


def _tc3(part, b2r):
    return pl.pallas_call(
        _tc3_body,
        grid=(N_NODES // _BN3,),
        in_specs=[
            pl.BlockSpec((2, _BN3, C2), lambda i: (0, i, 0)),
            pl.BlockSpec((1, C2), lambda i: (0, 0)),
        ],
        out_specs=pl.BlockSpec((_BN3, NCLS), lambda i: (i, 0)),
        out_shape=jax.ShapeDtypeStruct((N_NODES, NCLS), _f32),
    )(part, b2r)


def _message_pipeline(nsub, u0_base, fill, drain, wait_scatter):
    fill(u0_base + 0, 0)
    fill(u0_base + 1, 1)

    def quad(kk, _):
        l0 = 4 * kk
        for r in range(4):
            lu = l0 + r
            u = u0_base + lu
            b2 = (r + 2) % 4
            drain(u, r)

            @pl.when(lu >= 2)
            def _():
                wait_scatter(b2)

            @pl.when(lu + 2 < nsub)
            def _():
                fill(u + 2, b2)
        return 0
    lax.fori_loop(0, nsub // 4, quad, 0)
    wait_scatter(2)
    wait_scatter(3)


def _make_sc1(e_pad):
    ept = e_pad // NTILES
    nblk = ept // BLK
    mesh = plsc.VectorSubcoreMesh(core_axis_name="c", subcore_axis_name="s")

    def body(src_hbm, dst_hbm, h1_hbm, asad_hbm, z64_hbm, o1_hbm,
             t0, t1, rden, srcb, dstb,
             gib0, gib1, gib2, gib3, scb0, scb1, scb2, scb3,
             alb0, alb1, alb2, alb3, rows0, rows1, rows2, rows3,
             irb, dsh, acc,
             gs0, gs1, gs2, gs3, ss0, ss1, ss2, ss3):
        cid = lax.axis_index("c")
        sid = lax.axis_index("s")
        gb = (gib0, gib1, gib2, gib3)
        sb = (scb0, scb1, scb2, scb3)
        ab = (alb0, alb1, alb2, alb3)
        rb = (rows0, rows1, rows2, rows3)
        gs = (gs0, gs1, gs2, gs3)
        ss = (ss0, ss1, ss2, ss3)

        def mk_irb(i, _):
            irb[pl.ds(i * 16, 16)] = _iota16() + i * 16
            return 0
        lax.fori_loop(0, NR // 16, mk_irb, 0)

        for g in range(4):
            h0 = 4 * cid + g

            def zrden(i, _):
                for q in range(8):
                    rden[i, pl.ds(q * 16, 16)] = jnp.zeros((16,), _f32)
                return 0
            lax.fori_loop(0, NR, zrden, 0)

            @pl.when(sid == 0)
            def _():
                pltpu.sync_copy(rden, dsh)

            pltpu.sync_copy(asad_hbm.at[h0], t0)
            pltpu.sync_copy(asad_hbm.at[HEADS + h0], t1)
            pltpu.sync_copy(z64_hbm, acc.at[pl.ds(sid * SLAB, SLAB)])
            plsc.subcore_barrier()

            def blkA(bi, _):
                base = sid * ept + bi * BLK
                pltpu.sync_copy(src_hbm.at[pl.ds(base, BLK)], srcb)
                pltpu.sync_copy(dst_hbm.at[pl.ds(base, BLK)], dstb)

                def vgA(i, _):
                    s = srcb[pl.ds(i * 16, 16)]
                    d = dstb[pl.ds(i * 16, 16)]
                    e = plsc.load_gather(t0, [s]) + plsc.load_gather(t1, [d])
                    ex = jnp.exp(_leaky(e))
                    plsc.addupdate_scatter(rden, [d >> 7, d & 127], ex)
                    return 0
                lax.fori_loop(0, BLK // 16, vgA, 0)
                return 0
            lax.fori_loop(0, nblk, blkA, 0)

            pltpu.sync_copy(rden, dsh.at[irb], add=True)
            plsc.subcore_barrier()
            pltpu.sync_copy(dsh, rden)

            def recip(i, _):
                for q in range(8):
                    v = rden[i, pl.ds(q * 16, 16)]
                    rden[i, pl.ds(q * 16, 16)] = 1.0 / (v + 1e-16)
                return 0
            lax.fori_loop(0, NR, recip, 0)

            def fill(u, b):
                gib, scb, alb, rows = gb[b], sb[b], ab[b], rb[b]

                def vg(i, _):
                    s = srcb[pl.ds(u * CHUNK + i * 16, 16)]
                    d = dstb[pl.ds(u * CHUNK + i * 16, 16)]
                    e = plsc.load_gather(t0, [s]) + plsc.load_gather(t1, [d])
                    ex = jnp.exp(_leaky(e))
                    rd = plsc.load_gather(rden, [d >> 7, d & 127])
                    alb[pl.ds(i * 16, 16)] = ex * rd
                    gib[pl.ds(i * 16, 16)] = (s << 3) + h0
                    scb[pl.ds(i * 16, 16)] = d
                    return 0
                lax.fori_loop(0, CHUNK // 16, vg, 0)
                pltpu.async_copy(h1_hbm.at[gib], rows, gs[b])

            def drain(u, b):
                gib, scb, alb, rows = gb[b], sb[b], ab[b], rb[b]
                pltpu.make_async_copy(h1_hbm.at[gib], rows, gs[b]).wait()

                def scale(e3, _):
                    for r in range(8):
                        e2 = e3 * 8 + r
                        al = plsc.load_gather(
                            alb, [jnp.full((16,), e2, _i32)])
                        for q in range(HID // 16):
                            rows[e2, pl.ds(q * 16, 16)] = (
                                rows[e2, pl.ds(q * 16, 16)] * al)
                    return 0
                lax.fori_loop(0, CHUNK // 8, scale, 0)
                pltpu.async_copy(rows, acc.at[scb], ss[b], add=True)

            def wait_scatter(b):
                pltpu.make_async_copy(rb[b], acc.at[sb[b]], ss[b]).wait()

            def blkB(bi, _):
                base = sid * ept + bi * BLK
                pltpu.sync_copy(src_hbm.at[pl.ds(base, BLK)], srcb)
                pltpu.sync_copy(dst_hbm.at[pl.ds(base, BLK)], dstb)
                _message_pipeline(BLKSUB, 0, fill, drain, wait_scatter)
                return 0
            lax.fori_loop(0, nblk, blkB, 0)

            plsc.subcore_barrier()
            pltpu.sync_copy(acc.at[pl.ds(sid * SLAB, SLAB)],
                            o1_hbm.at[h0, pl.ds(sid * SLAB, SLAB)])
            plsc.subcore_barrier()

    return pl.kernel(
        body,
        out_type=jax.ShapeDtypeStruct((HEADS, N_PAD, HID), _f32),
        mesh=mesh,
        scratch_types=(
            [pltpu.VMEM((N_PAD,), _f32)] * 2
            + [pltpu.VMEM((NR, 128), _f32)]
            + [pltpu.VMEM((BLK,), _i32)] * 2
            + [pltpu.VMEM((CHUNK,), _i32)] * 8
            + [pltpu.VMEM((CHUNK,), _f32)] * 4
            + [pltpu.VMEM((CHUNK, HID), _f32)] * 4
            + [pltpu.VMEM((NR,), _i32)]
            + [pltpu.VMEM_SHARED((NR, 128), _f32)]
            + [pltpu.VMEM_SHARED((N_PAD, HID), _f32)]
            + [pltpu.SemaphoreType.DMA] * 8
        ),
        compiler_params=_SC_PARAMS,
    )


def _make_sc2(e_pad):
    ept = e_pad // NTILES
    nblk = ept // BLK
    blk_b = BLK // NSC
    bsub = blk_b // CHUNK
    mesh = plsc.VectorSubcoreMesh(core_axis_name="c", subcore_axis_name="s")

    def body(src_hbm, dst_hbm, h2_hbm, asad_hbm, z48_hbm, part_hbm,
             t0, t1, rden, srcb, dstb,
             gib0, gib1, gib2, gib3, scb0, scb1, scb2, scb3,
             alb0, alb1, alb2, alb3, rows0, rows1, rows2, rows3,
             irb, dsh, acc,
             gs0, gs1, gs2, gs3, ss0, ss1, ss2, ss3):
        cid = lax.axis_index("c")
        sid = lax.axis_index("s")
        gb = (gib0, gib1, gib2, gib3)
        sb = (scb0, scb1, scb2, scb3)
        ab = (alb0, alb1, alb2, alb3)
        rb = (rows0, rows1, rows2, rows3)
        gs = (gs0, gs1, gs2, gs3)
        ss = (ss0, ss1, ss2, ss3)

        def mk_irb(i, _):
            irb[pl.ds(i * 16, 16)] = _iota16() + i * 16
            return 0
        lax.fori_loop(0, NR // 16, mk_irb, 0)

        def zrden(i, _):
            for q in range(8):
                rden[i, pl.ds(q * 16, 16)] = jnp.zeros((16,), _f32)
            return 0
        lax.fori_loop(0, NR, zrden, 0)

        @pl.when(sid == 0)
        def _():
            pltpu.sync_copy(rden, dsh)

        pltpu.sync_copy(asad_hbm.at[0], t0)
        pltpu.sync_copy(asad_hbm.at[1], t1)
        pltpu.sync_copy(z48_hbm, acc.at[pl.ds(sid * SLAB, SLAB)])
        plsc.subcore_barrier()

        def blkA(bi, _):
            base = sid * ept + bi * BLK
            pltpu.sync_copy(src_hbm.at[pl.ds(base, BLK)], srcb)
            pltpu.sync_copy(dst_hbm.at[pl.ds(base, BLK)], dstb)

            def vgA(i, _):
                s = srcb[pl.ds(i * 16, 16)]
                d = dstb[pl.ds(i * 16, 16)]
                e = plsc.load_gather(t0, [s]) + plsc.load_gather(t1, [d])
                ex = jnp.exp(_leaky(e))
                plsc.addupdate_scatter(rden, [d >> 7, d & 127], ex)
                return 0
            lax.fori_loop(0, BLK // 16, vgA, 0)
            return 0
        lax.fori_loop(0, nblk, blkA, 0)

        pltpu.sync_copy(rden, dsh.at[irb], add=True)
        plsc.subcore_barrier()
        pltpu.sync_copy(dsh, rden)

        def recip(i, _):
            for q in range(8):
                v = rden[i, pl.ds(q * 16, 16)]
                rden[i, pl.ds(q * 16, 16)] = 1.0 / (v + 1e-16)
            return 0
        lax.fori_loop(0, NR, recip, 0)

        def fill(u, b):
            gib, scb, alb, rows = gb[b], sb[b], ab[b], rb[b]

            def vg(i, _):
                s = srcb[pl.ds(u * CHUNK + i * 16, 16)]
                d = dstb[pl.ds(u * CHUNK + i * 16, 16)]
                e = plsc.load_gather(t0, [s]) + plsc.load_gather(t1, [d])
                ex = jnp.exp(_leaky(e))
                rd = plsc.load_gather(rden, [d >> 7, d & 127])
                alb[pl.ds(i * 16, 16)] = ex * rd
                gib[pl.ds(i * 16, 16)] = s
                scb[pl.ds(i * 16, 16)] = d
                return 0
            lax.fori_loop(0, CHUNK // 16, vg, 0)
            pltpu.async_copy(h2_hbm.at[gib], rows, gs[b])

        def drain(u, b):
            gib, scb, alb, rows = gb[b], sb[b], ab[b], rb[b]
            pltpu.make_async_copy(h2_hbm.at[gib], rows, gs[b]).wait()

            def scale(e3, _):
                for r in range(8):
                    e2 = e3 * 8 + r
                    al = plsc.load_gather(alb, [jnp.full((16,), e2, _i32)])
                    for q in range(C2 // 16):
                        rows[e2, pl.ds(q * 16, 16)] = (
                            rows[e2, pl.ds(q * 16, 16)] * al)
                return 0
            lax.fori_loop(0, CHUNK // 8, scale, 0)
            pltpu.async_copy(rows, acc.at[scb], ss[b], add=True)

        def wait_scatter(b):
            pltpu.make_async_copy(rb[b], acc.at[sb[b]], ss[b]).wait()

        def blkB(bi, _):
            base = sid * ept + cid * (ept // NSC) + bi * blk_b
            pltpu.sync_copy(src_hbm.at[pl.ds(base, blk_b)],
                            srcb.at[pl.ds(0, blk_b)])
            pltpu.sync_copy(dst_hbm.at[pl.ds(base, blk_b)],
                            dstb.at[pl.ds(0, blk_b)])
            _message_pipeline(bsub, 0, fill, drain, wait_scatter)
            return 0
        lax.fori_loop(0, nblk, blkB, 0)

        plsc.subcore_barrier()
        pltpu.sync_copy(acc.at[pl.ds(sid * SLAB, SLAB)],
                        part_hbm.at[cid, pl.ds(sid * SLAB, SLAB)])
        plsc.subcore_barrier()

    return pl.kernel(
        body,
        out_type=jax.ShapeDtypeStruct((NSC, N_PAD, C2), _f32),
        mesh=mesh,
        scratch_types=(
            [pltpu.VMEM((N_PAD,), _f32)] * 2
            + [pltpu.VMEM((NR, 128), _f32)]
            + [pltpu.VMEM((BLK,), _i32)] * 2
            + [pltpu.VMEM((CHUNK,), _i32)] * 8
            + [pltpu.VMEM((CHUNK,), _f32)] * 4
            + [pltpu.VMEM((CHUNK, C2), _f32)] * 4
            + [pltpu.VMEM((NR,), _i32)]
            + [pltpu.VMEM_SHARED((NR, 128), _f32)]
            + [pltpu.VMEM_SHARED((N_PAD, C2), _f32)]
            + [pltpu.SemaphoreType.DMA] * 8
        ),
        compiler_params=_SC_PARAMS,
    )


def kernel(x, edge_index, W1, att_src1, att_dst1, b1,
           W2, att_src2, att_dst2, b2):
    n_edges = edge_index.shape[1]
    e_tot = n_edges + N_NODES
    unit = NTILES * BLK
    e_pad = ((e_tot + unit - 1) // unit) * unit

    loop = jnp.arange(N_NODES, dtype=_i32)
    padv = N_NODES + jnp.arange(e_pad - e_tot, dtype=_i32) % (N_PAD - N_NODES)
    src = jnp.concatenate([edge_index[0].astype(_i32), loop, padv])
    dst = jnp.concatenate([edge_index[1].astype(_i32), loop, padv])

    x_p = jnp.zeros((N_PAD, IN_DIM), _f32).at[:N_NODES].set(x)

    eye = jnp.eye(HEADS, dtype=_f32)
    as1 = (att_src1[:, :, None] * eye[:, None, :]).reshape(HEADS * HID, HEADS)
    ad1 = (att_dst1[:, :, None] * eye[:, None, :]).reshape(HEADS * HID, HEADS)
    aw1 = jnp.concatenate([as1, ad1], axis=1)

    h1, asad1 = _tc1(x_p, W1, aw1)
    h1v = h1.reshape(N_PAD * HEADS, HID)

    z64 = jnp.zeros((SLAB, HID), _f32)
    o1 = _make_sc1(e_pad)(src, dst, h1v, asad1, z64)

    b1r = b1.reshape(HEADS, HID)
    w2p = jnp.zeros((HEADS, HID, C2), _f32).at[:, :, :NCLS].set(
        W2.reshape(HEADS, HID, NCLS))
    a2 = jnp.zeros((2, C2), _f32)
    a2 = a2.at[0, :NCLS].set(att_src2[0]).at[1, :NCLS].set(att_dst2[0])

    h2, asad2 = _tc2(o1, b1r, w2p, a2)

    z48 = jnp.zeros((SLAB, C2), _f32)
    part = _make_sc2(e_pad)(src, dst, h2, asad2, z48)

    b2r = jnp.zeros((1, C2), _f32).at[0, :NCLS].set(b2)
    return _tc3(part, b2r)


# speedup vs baseline: 1.0192x; 1.0192x over previous
import functools

import jax
import jax.numpy as jnp
from jax import lax
from jax.experimental import pallas as pl
from jax.experimental.pallas import tpu as pltpu
from jax.experimental.pallas import tpu_sc as plsc

N_NODES = 10000
IN_DIM = 128
HID = 64
HEADS = 8
NCLS = 40

N_PAD = 10240
NR = N_PAD // 128
C2 = 48
CHUNK = 128
NSC = 2
NTILES = 16
SLAB = N_PAD // NTILES
BLKSUB = 56
BLK = BLKSUB * CHUNK
MCH = 64

_f32 = jnp.float32
_i32 = jnp.int32

_SC_PARAMS = pltpu.CompilerParams(
    needs_layout_passes=False, use_tc_tiling_on_sc=False)


def _iota16():
    return lax.iota(_i32, 16)


def _leaky(e):
    return jnp.maximum(e, 0.2 * e)


_BN1 = 1024


def _tc1_body(x_ref, w_ref, aw_ref, h_ref, asad_ref):
    h = jnp.dot(x_ref[...], w_ref[...], preferred_element_type=_f32)
    h_ref[...] = h
    asad_ref[...] = lax.dot_general(
        aw_ref[...], h, (((0,), (1,)), ((), ())), preferred_element_type=_f32)


def _tc1(x_p, w1, aw1):
    return pl.pallas_call(
        _tc1_body,
        grid=(N_PAD // _BN1,),
        in_specs=[
            pl.BlockSpec((_BN1, IN_DIM), lambda i: (i, 0)),
            pl.BlockSpec((IN_DIM, HEADS * HID), lambda i: (0, 0)),
            pl.BlockSpec((HEADS * HID, 2 * HEADS), lambda i: (0, 0)),
        ],
        out_specs=[
            pl.BlockSpec((_BN1, HEADS * HID), lambda i: (i, 0)),
            pl.BlockSpec((2 * HEADS, _BN1), lambda i: (0, i)),
        ],
        out_shape=[
            jax.ShapeDtypeStruct((N_PAD, HEADS * HID), _f32),
            jax.ShapeDtypeStruct((2 * HEADS, N_PAD), _f32),
        ],
    )(x_p, w1, aw1)


_BN2 = 1024


def _tc2_body(o1_ref, b1_ref, w2_ref, a2_ref, h2_ref, asad2_ref):
    acc = jnp.zeros((_BN2, C2), _f32)
    for h in range(HEADS):
        v = o1_ref[h] + b1_ref[h][None, :]
        emb_h = jnp.where(v > 0, v, jnp.exp(jnp.minimum(v, 0.0)) - 1.0)
        acc = acc + jnp.dot(emb_h, w2_ref[h], preferred_element_type=_f32)
    h2_ref[...] = acc
    a2 = a2_ref[...]
    s = jnp.sum(acc * a2[0][None, :], axis=1)
    d = jnp.sum(acc * a2[1][None, :], axis=1)
    asad2_ref[...] = jnp.concatenate([s[None, :], d[None, :]], axis=0)


def _tc2(o1, b1r, w2p, a2):
    return pl.pallas_call(
        _tc2_body,
        grid=(N_PAD // _BN2,),
        in_specs=[
            pl.BlockSpec((HEADS, _BN2, HID), lambda i: (0, i, 0)),
            pl.BlockSpec((HEADS, HID), lambda i: (0, 0)),
            pl.BlockSpec((HEADS, HID, C2), lambda i: (0, 0, 0)),
            pl.BlockSpec((2, C2), lambda i: (0, 0)),
        ],
        out_specs=[
            pl.BlockSpec((_BN2, C2), lambda i: (i, 0)),
            pl.BlockSpec((2, _BN2), lambda i: (0, i)),
        ],
        out_shape=[
            jax.ShapeDtypeStruct((N_PAD, C2), _f32),
            jax.ShapeDtypeStruct((2, N_PAD), _f32),
        ],
    )(o1, b1r, w2p, a2)


_BN3 = 2000


def _tc3_body(p_ref, b2_ref, out_ref):
    s = p_ref[0] + p_ref[1] + b2_ref[...]
    out_ref[...] = s[:, :NCLS]


def _tc3(part, b2r):
    return pl.pallas_call(
        _tc3_body,
        grid=(N_NODES // _BN3,),
        in_specs=[
            pl.BlockSpec((2, _BN3, C2), lambda i: (0, i, 0)),
            pl.BlockSpec((1, C2), lambda i: (0, 0)),
        ],
        out_specs=pl.BlockSpec((_BN3, NCLS), lambda i: (i, 0)),
        out_shape=jax.ShapeDtypeStruct((N_NODES, NCLS), _f32),
    )(part, b2r)


def _message_pipeline(nsub, u0_base, fill, drain, wait_scatter):
    fill(u0_base + 0, 0)
    fill(u0_base + 1, 1)

    def quad(kk, _):
        l0 = 4 * kk
        for r in range(4):
            lu = l0 + r
            u = u0_base + lu
            b2 = (r + 2) % 4
            drain(u, r)

            @pl.when(lu >= 2)
            def _():
                wait_scatter(b2)

            @pl.when(lu + 2 < nsub)
            def _():
                fill(u + 2, b2)
        return 0
    lax.fori_loop(0, nsub // 4, quad, 0)
    wait_scatter(2)
    wait_scatter(3)


def _make_sc1(e_pad):
    ept = e_pad // NTILES
    nblk = ept // BLK
    mesh = plsc.VectorSubcoreMesh(core_axis_name="c", subcore_axis_name="s")

    def body(src_hbm, dst_hbm, h1_hbm, asad_hbm, z64_hbm, o1_hbm,
             t0, t1, rden, srcb, dstb,
             gib0, gib1, gib2, gib3, scb0, scb1, scb2, scb3,
             alb0, alb1, alb2, alb3, rows0, rows1, rows2, rows3,
             irb, dsh, acc,
             gs0, gs1, gs2, gs3, ss0, ss1, ss2, ss3):
        cid = lax.axis_index("c")
        sid = lax.axis_index("s")
        gb = (gib0, gib1, gib2, gib3)
        sb = (scb0, scb1, scb2, scb3)
        ab = (alb0, alb1, alb2, alb3)
        rb = (rows0, rows1, rows2, rows3)
        gs = (gs0, gs1, gs2, gs3)
        ss = (ss0, ss1, ss2, ss3)

        def mk_irb(i, _):
            irb[pl.ds(i * 16, 16)] = _iota16() + i * 16
            return 0
        lax.fori_loop(0, NR // 16, mk_irb, 0)

        for g in range(4):
            h0 = 4 * cid + g

            def zrden(i, _):
                for q in range(8):
                    rden[i, pl.ds(q * 16, 16)] = jnp.zeros((16,), _f32)
                return 0
            lax.fori_loop(0, NR, zrden, 0)

            @pl.when(sid == 0)
            def _():
                pltpu.sync_copy(rden, dsh)

            pltpu.sync_copy(asad_hbm.at[h0], t0)
            pltpu.sync_copy(asad_hbm.at[HEADS + h0], t1)
            pltpu.sync_copy(z64_hbm, acc.at[pl.ds(sid * SLAB, SLAB)])
            plsc.subcore_barrier()

            def blkA(bi, _):
                base = sid * ept + bi * BLK
                pltpu.sync_copy(src_hbm.at[pl.ds(base, BLK)], srcb)
                pltpu.sync_copy(dst_hbm.at[pl.ds(base, BLK)], dstb)

                def vgA(i, _):
                    s = srcb[pl.ds(i * 16, 16)]
                    d = dstb[pl.ds(i * 16, 16)]
                    e = plsc.load_gather(t0, [s]) + plsc.load_gather(t1, [d])
                    ex = jnp.exp(_leaky(e))
                    plsc.addupdate_scatter(rden, [d >> 7, d & 127], ex)
                    return 0
                lax.fori_loop(0, BLK // 16, vgA, 0)
                return 0
            lax.fori_loop(0, nblk, blkA, 0)

            pltpu.sync_copy(rden, dsh.at[irb], add=True)
            plsc.subcore_barrier()
            pltpu.sync_copy(dsh, rden)

            def recip(i, _):
                for q in range(8):
                    v = rden[i, pl.ds(q * 16, 16)]
                    rden[i, pl.ds(q * 16, 16)] = 1.0 / (v + 1e-16)
                return 0
            lax.fori_loop(0, NR, recip, 0)

            def fill(u, b):
                gib, scb, alb, rows = gb[b], sb[b], ab[b], rb[b]

                def vg(i, _):
                    s = srcb[pl.ds(u * CHUNK + i * 16, 16)]
                    d = dstb[pl.ds(u * CHUNK + i * 16, 16)]
                    e = plsc.load_gather(t0, [s]) + plsc.load_gather(t1, [d])
                    ex = jnp.exp(_leaky(e))
                    rd = plsc.load_gather(rden, [d >> 7, d & 127])
                    alb[pl.ds(i * 16, 16)] = ex * rd
                    gib[pl.ds(i * 16, 16)] = (s << 3) + h0
                    scb[pl.ds(i * 16, 16)] = d
                    return 0
                lax.fori_loop(0, CHUNK // 16, vg, 0)
                pltpu.async_copy(h1_hbm.at[gib], rows, gs[b])

            def drain(u, b):
                gib, scb, alb, rows = gb[b], sb[b], ab[b], rb[b]
                pltpu.make_async_copy(h1_hbm.at[gib], rows, gs[b]).wait()

                def scale(e3, _):
                    for r in range(4):
                        e2 = e3 * 4 + r
                        al = plsc.load_gather(
                            alb, [jnp.full((16,), e2, _i32)])
                        for q in range(HID // 16):
                            rows[e2, pl.ds(q * 16, 16)] = (
                                rows[e2, pl.ds(q * 16, 16)] * al)
                    return 0
                lax.fori_loop(0, CHUNK // 4, scale, 0)
                pltpu.async_copy(rows, acc.at[scb], ss[b], add=True)

            def wait_scatter(b):
                pltpu.make_async_copy(rb[b], acc.at[sb[b]], ss[b]).wait()

            def blkB(bi, _):
                base = sid * ept + bi * BLK
                pltpu.sync_copy(src_hbm.at[pl.ds(base, BLK)], srcb)
                pltpu.sync_copy(dst_hbm.at[pl.ds(base, BLK)], dstb)
                _message_pipeline(BLKSUB, 0, fill, drain, wait_scatter)
                return 0
            lax.fori_loop(0, nblk, blkB, 0)

            plsc.subcore_barrier()
            pltpu.sync_copy(acc.at[pl.ds(sid * SLAB, SLAB)],
                            o1_hbm.at[h0, pl.ds(sid * SLAB, SLAB)])
            plsc.subcore_barrier()

    return pl.kernel(
        body,
        out_type=jax.ShapeDtypeStruct((HEADS, N_PAD, HID), _f32),
        mesh=mesh,
        scratch_types=(
            [pltpu.VMEM((N_PAD,), _f32)] * 2
            + [pltpu.VMEM((NR, 128), _f32)]
            + [pltpu.VMEM((BLK,), _i32)] * 2
            + [pltpu.VMEM((CHUNK,), _i32)] * 8
            + [pltpu.VMEM((CHUNK,), _f32)] * 4
            + [pltpu.VMEM((CHUNK, HID), _f32)] * 4
            + [pltpu.VMEM((NR,), _i32)]
            + [pltpu.VMEM_SHARED((NR, 128), _f32)]
            + [pltpu.VMEM_SHARED((N_PAD, HID), _f32)]
            + [pltpu.SemaphoreType.DMA] * 8
        ),
        compiler_params=_SC_PARAMS,
    )


def _make_sc2(e_pad):
    ept = e_pad // NTILES
    nblk = ept // BLK
    blk_b = BLK // NSC
    bsub = blk_b // CHUNK
    mesh = plsc.VectorSubcoreMesh(core_axis_name="c", subcore_axis_name="s")

    def body(src_hbm, dst_hbm, h2_hbm, asad_hbm, z48_hbm, part_hbm,
             t0, t1, rden, srcb, dstb,
             gib0, gib1, gib2, gib3, scb0, scb1, scb2, scb3,
             alb0, alb1, alb2, alb3, rows0, rows1, rows2, rows3,
             irb, dsh, acc,
             gs0, gs1, gs2, gs3, ss0, ss1, ss2, ss3):
        cid = lax.axis_index("c")
        sid = lax.axis_index("s")
        gb = (gib0, gib1, gib2, gib3)
        sb = (scb0, scb1, scb2, scb3)
        ab = (alb0, alb1, alb2, alb3)
        rb = (rows0, rows1, rows2, rows3)
        gs = (gs0, gs1, gs2, gs3)
        ss = (ss0, ss1, ss2, ss3)

        def mk_irb(i, _):
            irb[pl.ds(i * 16, 16)] = _iota16() + i * 16
            return 0
        lax.fori_loop(0, NR // 16, mk_irb, 0)

        def zrden(i, _):
            for q in range(8):
                rden[i, pl.ds(q * 16, 16)] = jnp.zeros((16,), _f32)
            return 0
        lax.fori_loop(0, NR, zrden, 0)

        @pl.when(sid == 0)
        def _():
            pltpu.sync_copy(rden, dsh)

        pltpu.sync_copy(asad_hbm.at[0], t0)
        pltpu.sync_copy(asad_hbm.at[1], t1)
        pltpu.sync_copy(z48_hbm, acc.at[pl.ds(sid * SLAB, SLAB)])
        plsc.subcore_barrier()

        def blkA(bi, _):
            base = sid * ept + bi * BLK
            pltpu.sync_copy(src_hbm.at[pl.ds(base, BLK)], srcb)
            pltpu.sync_copy(dst_hbm.at[pl.ds(base, BLK)], dstb)

            def vgA(i, _):
                s = srcb[pl.ds(i * 16, 16)]
                d = dstb[pl.ds(i * 16, 16)]
                e = plsc.load_gather(t0, [s]) + plsc.load_gather(t1, [d])
                ex = jnp.exp(_leaky(e))
                plsc.addupdate_scatter(rden, [d >> 7, d & 127], ex)
                return 0
            lax.fori_loop(0, BLK // 16, vgA, 0)
            return 0
        lax.fori_loop(0, nblk, blkA, 0)

        pltpu.sync_copy(rden, dsh.at[irb], add=True)
        plsc.subcore_barrier()
        pltpu.sync_copy(dsh, rden)

        def recip(i, _):
            for q in range(8):
                v = rden[i, pl.ds(q * 16, 16)]
                rden[i, pl.ds(q * 16, 16)] = 1.0 / (v + 1e-16)
            return 0
        lax.fori_loop(0, NR, recip, 0)

        def fill(u, b):
            gib, scb, alb, rows = gb[b], sb[b], ab[b], rb[b]

            def vg(i, _):
                s = srcb[pl.ds(u * CHUNK + i * 16, 16)]
                d = dstb[pl.ds(u * CHUNK + i * 16, 16)]
                e = plsc.load_gather(t0, [s]) + plsc.load_gather(t1, [d])
                ex = jnp.exp(_leaky(e))
                rd = plsc.load_gather(rden, [d >> 7, d & 127])
                alb[pl.ds(i * 16, 16)] = ex * rd
                gib[pl.ds(i * 16, 16)] = s
                scb[pl.ds(i * 16, 16)] = d
                return 0
            lax.fori_loop(0, CHUNK // 16, vg, 0)
            pltpu.async_copy(h2_hbm.at[gib], rows, gs[b])

        def drain(u, b):
            gib, scb, alb, rows = gb[b], sb[b], ab[b], rb[b]
            pltpu.make_async_copy(h2_hbm.at[gib], rows, gs[b]).wait()

            def scale(e3, _):
                for r in range(4):
                    e2 = e3 * 4 + r
                    al = plsc.load_gather(alb, [jnp.full((16,), e2, _i32)])
                    for q in range(C2 // 16):
                        rows[e2, pl.ds(q * 16, 16)] = (
                            rows[e2, pl.ds(q * 16, 16)] * al)
                return 0
            lax.fori_loop(0, CHUNK // 4, scale, 0)
            pltpu.async_copy(rows, acc.at[scb], ss[b], add=True)

        def wait_scatter(b):
            pltpu.make_async_copy(rb[b], acc.at[sb[b]], ss[b]).wait()

        def blkB(bi, _):
            base = sid * ept + cid * (ept // NSC) + bi * blk_b
            pltpu.sync_copy(src_hbm.at[pl.ds(base, blk_b)],
                            srcb.at[pl.ds(0, blk_b)])
            pltpu.sync_copy(dst_hbm.at[pl.ds(base, blk_b)],
                            dstb.at[pl.ds(0, blk_b)])
            _message_pipeline(bsub, 0, fill, drain, wait_scatter)
            return 0
        lax.fori_loop(0, nblk, blkB, 0)

        plsc.subcore_barrier()
        pltpu.sync_copy(acc.at[pl.ds(sid * SLAB, SLAB)],
                        part_hbm.at[cid, pl.ds(sid * SLAB, SLAB)])
        plsc.subcore_barrier()

    return pl.kernel(
        body,
        out_type=jax.ShapeDtypeStruct((NSC, N_PAD, C2), _f32),
        mesh=mesh,
        scratch_types=(
            [pltpu.VMEM((N_PAD,), _f32)] * 2
            + [pltpu.VMEM((NR, 128), _f32)]
            + [pltpu.VMEM((BLK,), _i32)] * 2
            + [pltpu.VMEM((CHUNK,), _i32)] * 8
            + [pltpu.VMEM((CHUNK,), _f32)] * 4
            + [pltpu.VMEM((CHUNK, C2), _f32)] * 4
            + [pltpu.VMEM((NR,), _i32)]
            + [pltpu.VMEM_SHARED((NR, 128), _f32)]
            + [pltpu.VMEM_SHARED((N_PAD, C2), _f32)]
            + [pltpu.SemaphoreType.DMA] * 8
        ),
        compiler_params=_SC_PARAMS,
    )


def kernel(x, edge_index, W1, att_src1, att_dst1, b1,
           W2, att_src2, att_dst2, b2):
    n_edges = edge_index.shape[1]
    e_tot = n_edges + N_NODES
    unit = NTILES * BLK
    e_pad = ((e_tot + unit - 1) // unit) * unit

    loop = jnp.arange(N_NODES, dtype=_i32)
    padv = N_NODES + jnp.arange(e_pad - e_tot, dtype=_i32) % (N_PAD - N_NODES)
    src = jnp.concatenate([edge_index[0].astype(_i32), loop, padv])
    dst = jnp.concatenate([edge_index[1].astype(_i32), loop, padv])

    x_p = jnp.zeros((N_PAD, IN_DIM), _f32).at[:N_NODES].set(x)

    eye = jnp.eye(HEADS, dtype=_f32)
    as1 = (att_src1[:, :, None] * eye[:, None, :]).reshape(HEADS * HID, HEADS)
    ad1 = (att_dst1[:, :, None] * eye[:, None, :]).reshape(HEADS * HID, HEADS)
    aw1 = jnp.concatenate([as1, ad1], axis=1)

    h1, asad1 = _tc1(x_p, W1, aw1)
    h1v = h1.reshape(N_PAD * HEADS, HID)

    z64 = jnp.zeros((SLAB, HID), _f32)
    o1 = _make_sc1(e_pad)(src, dst, h1v, asad1, z64)

    b1r = b1.reshape(HEADS, HID)
    w2p = jnp.zeros((HEADS, HID, C2), _f32).at[:, :, :NCLS].set(
        W2.reshape(HEADS, HID, NCLS))
    a2 = jnp.zeros((2, C2), _f32)
    a2 = a2.at[0, :NCLS].set(att_src2[0]).at[1, :NCLS].set(att_dst2[0])

    h2, asad2 = _tc2(o1, b1r, w2p, a2)

    z48 = jnp.zeros((SLAB, C2), _f32)
    part = _make_sc2(e_pad)(src, dst, h2, asad2, z48)

    b2r = jnp.zeros((1, C2), _f32).at[0, :NCLS].set(b2)
    return _tc3(part, b2r)
